# Initial kernel scaffold; baseline (speedup 1.0000x reference)
import jax
import jax.numpy as jnp
from jax.experimental import pallas as pl


def kernel(hidden_states, g, weight_token, weight_gene, Wg, Wu, Wd, Wsg, Wsu, Wsd):
    raise NotImplementedError("write your pallas kernel here")


# speedup vs baseline: 4.4358x; 4.4358x over previous
import functools

import jax
import jax.numpy as jnp
from jax.experimental import pallas as pl
from jax.experimental.pallas import tpu as pltpu

B, S, H = 1, 2048, 1024
E, K = 8, 2
I = 256
G = 128
TEMP = 1.0
NE = E + 1


def _moe_body(xbf_ref, g_ref, wt_ref, wgene_ref,
              Wg_ref, Wu_ref, Wd_ref, out_ref, w9_scr):
    e = pl.program_id(0)

    @pl.when(e == 0)
    def _router():
        dn = (((1,), (1,)), ((), ()))
        logits_h = jax.lax.dot_general(
            xbf_ref[...], wt_ref[...], dn,
            preferred_element_type=jnp.float32)
        logits_g = jax.lax.dot_general(
            g_ref[...], wgene_ref[...], dn,
            preferred_element_type=jnp.float32)
        logits = (logits_h + logits_g / TEMP) / (1.0 + 1.0 / TEMP)
        m = jnp.max(logits, axis=1, keepdims=True)
        ex = jnp.exp(logits - m)
        scores = ex / jnp.sum(ex, axis=1, keepdims=True)
        lane = jax.lax.broadcasted_iota(jnp.int32, (S, E), 1)
        m1 = jnp.max(scores, axis=1, keepdims=True)
        i1 = jnp.min(jnp.where(scores == m1, lane, E), axis=1, keepdims=True)
        masked = jnp.where(lane == i1, -jnp.inf, scores)
        m2 = jnp.max(masked, axis=1, keepdims=True)
        i2 = jnp.min(jnp.where(masked == m2, lane, E), axis=1, keepdims=True)
        denom = m1 + m2 + 1e-20
        w1 = m1 / denom
        w2 = m2 / denom
        lane16 = jax.lax.broadcasted_iota(jnp.int32, (S, 16), 1)
        w9 = (jnp.where(lane16 == i1, w1, 0.0)
              + jnp.where(lane16 == i2, w2, 0.0)
              + (lane16 == E).astype(jnp.float32))
        w9_scr[...] = w9

    lane16 = jax.lax.broadcasted_iota(jnp.int32, (S, 16), 1)
    wcol = jnp.sum(jnp.where(lane16 == e, w9_scr[...], 0.0),
                   axis=1, keepdims=True)

    dn = (((1,), (1,)), ((), ()))
    xb = xbf_ref[...]
    xg = jax.lax.dot_general(xb, Wg_ref[0], dn,
                             preferred_element_type=jnp.float32)
    xu = jax.lax.dot_general(xb, Wu_ref[0], dn,
                             preferred_element_type=jnp.float32)
    h = jnp.where(xg >= 0, xg, 0.01 * xg) * xu
    y = jax.lax.dot_general(h.astype(jnp.bfloat16), Wd_ref[0], dn,
                            preferred_element_type=jnp.float32)
    y = y * wcol

    @pl.when(e == 0)
    def _init():
        out_ref[...] = y

    @pl.when(e != 0)
    def _acc():
        out_ref[...] += y


@jax.jit
def kernel(hidden_states, g, weight_token, weight_gene,
           Wg, Wu, Wd, Wsg, Wsu, Wsd):
    xbf = hidden_states.reshape(S, H).astype(jnp.bfloat16)
    g_bf = g.astype(jnp.bfloat16)
    wt_bf = weight_token.astype(jnp.bfloat16)
    wgene_bf = weight_gene.astype(jnp.bfloat16)
    Wg_all = jnp.concatenate([Wg, Wsg[None]], axis=0).astype(jnp.bfloat16)
    Wu_all = jnp.concatenate([Wu, Wsu[None]], axis=0).astype(jnp.bfloat16)
    Wd_all = jnp.concatenate([Wd, Wsd[None]], axis=0).astype(jnp.bfloat16)

    y = pl.pallas_call(
        _moe_body,
        grid=(NE,),
        in_specs=[
            pl.BlockSpec((S, H), lambda e: (0, 0)),
            pl.BlockSpec((1, G), lambda e: (0, 0)),
            pl.BlockSpec((E, H), lambda e: (0, 0)),
            pl.BlockSpec((E, G), lambda e: (0, 0)),
            pl.BlockSpec((1, I, H), lambda e: (e, 0, 0)),
            pl.BlockSpec((1, I, H), lambda e: (e, 0, 0)),
            pl.BlockSpec((1, H, I), lambda e: (e, 0, 0)),
        ],
        out_specs=pl.BlockSpec((S, H), lambda e: (0, 0)),
        out_shape=jax.ShapeDtypeStruct((S, H), jnp.float32),
        scratch_shapes=[pltpu.VMEM((S, 16), jnp.float32)],
        compiler_params=pltpu.CompilerParams(
            dimension_semantics=("arbitrary",)),
    )(xbf, g_bf, wt_bf, wgene_bf, Wg_all, Wu_all, Wd_all)
    return y.reshape(B, S, H)


# speedup vs baseline: 4.6724x; 1.0533x over previous
import jax
import jax.numpy as jnp
from jax.experimental import pallas as pl
from jax.experimental.pallas import tpu as pltpu

B, S, H = 1, 2048, 1024
E, K = 8, 2
I = 256
G = 128
TEMP = 1.0
NE = E + 1


def _moe_body(xbf_ref, g_ref, wt_ref, wgene_ref,
              Wg_ref, Wu_ref, Wd_ref, out_ref):
    dn = (((1,), (1,)), ((), ()))
    xb = xbf_ref[...]

    logits_h = jax.lax.dot_general(
        xb, wt_ref[...], dn, preferred_element_type=jnp.float32)
    logits_g = jax.lax.dot_general(
        g_ref[...], wgene_ref[...], dn,
        preferred_element_type=jnp.float32)
    logits = (logits_h + logits_g / TEMP) / (1.0 + 1.0 / TEMP)
    m = jnp.max(logits, axis=1, keepdims=True)
    ex = jnp.exp(logits - m)
    scores = ex / jnp.sum(ex, axis=1, keepdims=True)
    lane = jax.lax.broadcasted_iota(jnp.int32, (S, E), 1)
    m1 = jnp.max(scores, axis=1, keepdims=True)
    i1 = jnp.min(jnp.where(scores == m1, lane, E), axis=1, keepdims=True)
    masked = jnp.where(lane == i1, -jnp.inf, scores)
    m2 = jnp.max(masked, axis=1, keepdims=True)
    i2 = jnp.min(jnp.where(masked == m2, lane, E), axis=1, keepdims=True)
    denom = m1 + m2 + 1e-20
    w1 = m1 / denom
    w2 = m2 / denom

    hs = []
    for e in range(NE):
        if e < E:
            wcol = (jnp.where(i1 == e, w1, 0.0)
                    + jnp.where(i2 == e, w2, 0.0))
        xg = jax.lax.dot_general(xb, Wg_ref[e], dn,
                                 preferred_element_type=jnp.float32)
        xu = jax.lax.dot_general(xb, Wu_ref[e], dn,
                                 preferred_element_type=jnp.float32)
        h = jnp.where(xg >= 0, xg, 0.01 * xg) * xu
        if e < E:
            h = h * wcol
        hs.append(h.astype(jnp.bfloat16))

    hall = jnp.concatenate(hs, axis=1)
    y = jax.lax.dot_general(hall, Wd_ref[...], (((1,), (0,)), ((), ())),
                            preferred_element_type=jnp.float32)
    out_ref[...] = y


@jax.jit
def kernel(hidden_states, g, weight_token, weight_gene,
           Wg, Wu, Wd, Wsg, Wsu, Wsd):
    xbf = hidden_states.reshape(S, H).astype(jnp.bfloat16)
    g_bf = g.astype(jnp.bfloat16)
    wt_bf = weight_token.astype(jnp.bfloat16)
    wgene_bf = weight_gene.astype(jnp.bfloat16)
    Wg_all = jnp.concatenate([Wg, Wsg[None]], axis=0).astype(jnp.bfloat16)
    Wu_all = jnp.concatenate([Wu, Wsu[None]], axis=0).astype(jnp.bfloat16)
    Wd_flat = (jnp.concatenate([Wd, Wsd[None]], axis=0)
               .transpose(0, 2, 1).reshape(NE * I, H).astype(jnp.bfloat16))

    y = pl.pallas_call(
        _moe_body,
        in_specs=[
            pl.BlockSpec((S, H), lambda: (0, 0)),
            pl.BlockSpec((1, G), lambda: (0, 0)),
            pl.BlockSpec((E, H), lambda: (0, 0)),
            pl.BlockSpec((E, G), lambda: (0, 0)),
            pl.BlockSpec((NE, I, H), lambda: (0, 0, 0)),
            pl.BlockSpec((NE, I, H), lambda: (0, 0, 0)),
            pl.BlockSpec((NE * I, H), lambda: (0, 0)),
        ],
        out_specs=pl.BlockSpec((S, H), lambda: (0, 0)),
        out_shape=jax.ShapeDtypeStruct((S, H), jnp.float32),
    )(xbf, g_bf, wt_bf, wgene_bf, Wg_all, Wu_all, Wd_flat)
    return y.reshape(B, S, H)


# speedup vs baseline: 6.0006x; 1.2843x over previous
import jax
import jax.numpy as jnp
from jax.experimental import pallas as pl
from jax.experimental.pallas import tpu as pltpu

B, S, H = 1, 2048, 1024
E, K = 8, 2
I = 256
G = 128
TEMP = 1.0
NE = E + 1


def _moe_body(x_ref, g_ref, wt_ref, wgene_ref, Wg_ref, Wu_ref, Wd_ref,
              Wsg_ref, Wsu_ref, Wsd_ref, out_ref):
    dn = (((1,), (1,)), ((), ()))
    xb = x_ref[...]

    logits_h = jax.lax.dot_general(
        xb, wt_ref[...].astype(jnp.bfloat16), dn,
        preferred_element_type=jnp.float32)
    logits_g = jax.lax.dot_general(
        g_ref[...].astype(jnp.bfloat16),
        wgene_ref[...].astype(jnp.bfloat16), dn,
        preferred_element_type=jnp.float32)
    logits = (logits_h + logits_g / TEMP) / (1.0 + 1.0 / TEMP)
    m = jnp.max(logits, axis=1, keepdims=True)
    ex = jnp.exp(logits - m)
    scores = ex / jnp.sum(ex, axis=1, keepdims=True)
    lane = jax.lax.broadcasted_iota(jnp.int32, (S, E), 1)
    m1 = jnp.max(scores, axis=1, keepdims=True)
    i1 = jnp.min(jnp.where(scores == m1, lane, E), axis=1, keepdims=True)
    masked = jnp.where(lane == i1, -jnp.inf, scores)
    m2 = jnp.max(masked, axis=1, keepdims=True)
    i2 = jnp.min(jnp.where(masked == m2, lane, E), axis=1, keepdims=True)
    denom = m1 + m2 + 1e-20
    w1 = m1 / denom
    w2 = m2 / denom

    for e in range(NE):
        if e < E:
            wg = Wg_ref[e].astype(jnp.bfloat16)
            wu = Wu_ref[e].astype(jnp.bfloat16)
            wd = Wd_ref[e].astype(jnp.bfloat16)
        else:
            wg = Wsg_ref[...].astype(jnp.bfloat16)
            wu = Wsu_ref[...].astype(jnp.bfloat16)
            wd = Wsd_ref[...].astype(jnp.bfloat16)
        xg = jax.lax.dot_general(xb, wg, dn,
                                 preferred_element_type=jnp.float32)
        xu = jax.lax.dot_general(xb, wu, dn,
                                 preferred_element_type=jnp.float32)
        h = jnp.where(xg >= 0, xg, 0.01 * xg) * xu
        if e < E:
            wcol = (jnp.where(i1 == e, w1, 0.0)
                    + jnp.where(i2 == e, w2, 0.0))
            h = h * wcol
        ye = jax.lax.dot_general(h.astype(jnp.bfloat16), wd, dn,
                                 preferred_element_type=jnp.float32)
        if e == 0:
            out_ref[...] = ye
        else:
            out_ref[...] += ye


@jax.jit
def kernel(hidden_states, g, weight_token, weight_gene,
           Wg, Wu, Wd, Wsg, Wsu, Wsd):
    y = pl.pallas_call(
        _moe_body,
        in_specs=[
            pl.BlockSpec((S, H), lambda: (0, 0)),
            pl.BlockSpec((1, G), lambda: (0, 0)),
            pl.BlockSpec((E, H), lambda: (0, 0)),
            pl.BlockSpec((E, G), lambda: (0, 0)),
            pl.BlockSpec((E, I, H), lambda: (0, 0, 0)),
            pl.BlockSpec((E, I, H), lambda: (0, 0, 0)),
            pl.BlockSpec((E, H, I), lambda: (0, 0, 0)),
            pl.BlockSpec((I, H), lambda: (0, 0)),
            pl.BlockSpec((I, H), lambda: (0, 0)),
            pl.BlockSpec((H, I), lambda: (0, 0)),
        ],
        out_specs=pl.BlockSpec((S, H), lambda: (0, 0)),
        out_shape=jax.ShapeDtypeStruct((S, H), jnp.float32),
    )(hidden_states.reshape(S, H).astype(jnp.bfloat16), g,
      weight_token, weight_gene, Wg, Wu, Wd, Wsg, Wsu, Wsd)
    return y.reshape(B, S, H)


# speedup vs baseline: 6.7990x; 1.1331x over previous
import jax
import jax.numpy as jnp
from jax.experimental import pallas as pl
from jax.experimental.pallas import tpu as pltpu

B, S, H = 1, 2048, 1024
E, K = 8, 2
I = 256
G = 128
TEMP = 1.0

_DN = (((1,), (1,)), ((), ()))


def _dmlp(x, wg, wu):
    xg = jax.lax.dot_general(x, wg, _DN, preferred_element_type=jnp.float32)
    xu = jax.lax.dot_general(x, wu, _DN, preferred_element_type=jnp.float32)
    return jnp.where(xg >= 0, xg, 0.01 * xg) * xu


def _moe_body(x_ref, g_ref, wt_ref, wgene_ref, Wsg_ref, Wsu_ref, Wsd_ref,
              Wg_ref, Wu_ref, Wd_ref, out_ref, w9_scr):
    e = pl.program_id(0)
    x = x_ref[...]

    @pl.when(e == 0)
    def _router_and_shared():
        logits_h = jax.lax.dot_general(
            x, wt_ref[...], _DN, preferred_element_type=jnp.float32)
        logits_g = jax.lax.dot_general(
            g_ref[...], wgene_ref[...], _DN,
            preferred_element_type=jnp.float32)
        logits = (logits_h + logits_g / TEMP) / (1.0 + 1.0 / TEMP)
        m = jnp.max(logits, axis=1, keepdims=True)
        ex = jnp.exp(logits - m)
        scores = ex / jnp.sum(ex, axis=1, keepdims=True)
        lane = jax.lax.broadcasted_iota(jnp.int32, (S, E), 1)
        m1 = jnp.max(scores, axis=1, keepdims=True)
        i1 = jnp.min(jnp.where(scores == m1, lane, E), axis=1, keepdims=True)
        masked = jnp.where(lane == i1, -jnp.inf, scores)
        m2 = jnp.max(masked, axis=1, keepdims=True)
        i2 = jnp.min(jnp.where(masked == m2, lane, E), axis=1, keepdims=True)
        denom = m1 + m2 + 1e-20
        w1 = m1 / denom
        w2 = m2 / denom
        lane16 = jax.lax.broadcasted_iota(jnp.int32, (S, 16), 1)
        w9_scr[...] = (jnp.where(lane16 == i1, w1, 0.0)
                       + jnp.where(lane16 == i2, w2, 0.0))

        hsh = _dmlp(x, Wsg_ref[...], Wsu_ref[...])
        out_ref[...] = jax.lax.dot_general(
            hsh, Wsd_ref[...], _DN, preferred_element_type=jnp.float32)

    lane16 = jax.lax.broadcasted_iota(jnp.int32, (S, 16), 1)
    wcol = jnp.sum(jnp.where(lane16 == e, w9_scr[...], 0.0),
                   axis=1, keepdims=True)
    h = _dmlp(x, Wg_ref[0], Wu_ref[0]) * wcol
    out_ref[...] += jax.lax.dot_general(
        h, Wd_ref[0], _DN, preferred_element_type=jnp.float32)


@jax.jit
def kernel(hidden_states, g, weight_token, weight_gene,
           Wg, Wu, Wd, Wsg, Wsu, Wsd):
    y = pl.pallas_call(
        _moe_body,
        grid=(E,),
        in_specs=[
            pl.BlockSpec((S, H), lambda e: (0, 0)),
            pl.BlockSpec((1, G), lambda e: (0, 0)),
            pl.BlockSpec((E, H), lambda e: (0, 0)),
            pl.BlockSpec((E, G), lambda e: (0, 0)),
            pl.BlockSpec((I, H), lambda e: (0, 0)),
            pl.BlockSpec((I, H), lambda e: (0, 0)),
            pl.BlockSpec((H, I), lambda e: (0, 0)),
            pl.BlockSpec((1, I, H), lambda e: (e, 0, 0)),
            pl.BlockSpec((1, I, H), lambda e: (e, 0, 0)),
            pl.BlockSpec((1, H, I), lambda e: (e, 0, 0)),
        ],
        out_specs=pl.BlockSpec((S, H), lambda e: (0, 0)),
        out_shape=jax.ShapeDtypeStruct((S, H), jnp.float32),
        scratch_shapes=[pltpu.VMEM((S, 16), jnp.float32)],
        compiler_params=pltpu.CompilerParams(
            dimension_semantics=("arbitrary",)),
    )(hidden_states.reshape(S, H), g, weight_token, weight_gene,
      Wsg, Wsu, Wsd, Wg, Wu, Wd)
    return y.reshape(B, S, H)


# speedup vs baseline: 7.2489x; 1.0662x over previous
import jax
import jax.numpy as jnp
from jax.experimental import pallas as pl
from jax.experimental.pallas import tpu as pltpu

B, S, H = 1, 2048, 1024
E, K = 8, 2
I = 256
G = 128
TEMP = 1.0

_DN = (((1,), (1,)), ((), ()))


def _dmlp(x, wg, wu):
    xg = jax.lax.dot_general(x, wg, _DN, preferred_element_type=jnp.float32)
    xu = jax.lax.dot_general(x, wu, _DN, preferred_element_type=jnp.float32)
    return jnp.where(xg >= 0, xg, 0.01 * xg) * xu


def _moe_body(x_ref, g_ref, wt_ref, wgene_ref, Wsg_ref, Wsu_ref, Wsd_ref,
              Wg_ref, Wu_ref, Wd_ref, out_ref, w9_scr, xb_scr):
    e = pl.program_id(0)

    @pl.when(e == 0)
    def _router_and_shared():
        xb_scr[...] = x_ref[...].astype(jnp.bfloat16)
        x = xb_scr[...]
        logits_h = jax.lax.dot_general(
            x, wt_ref[...].astype(jnp.bfloat16), _DN,
            preferred_element_type=jnp.float32)
        logits_g = jax.lax.dot_general(
            g_ref[...].astype(jnp.bfloat16),
            wgene_ref[...].astype(jnp.bfloat16), _DN,
            preferred_element_type=jnp.float32)
        logits = (logits_h + logits_g / TEMP) / (1.0 + 1.0 / TEMP)
        m = jnp.max(logits, axis=1, keepdims=True)
        ex = jnp.exp(logits - m)
        scores = ex / jnp.sum(ex, axis=1, keepdims=True)
        lane = jax.lax.broadcasted_iota(jnp.int32, (S, E), 1)
        m1 = jnp.max(scores, axis=1, keepdims=True)
        i1 = jnp.min(jnp.where(scores == m1, lane, E), axis=1, keepdims=True)
        masked = jnp.where(lane == i1, -jnp.inf, scores)
        m2 = jnp.max(masked, axis=1, keepdims=True)
        i2 = jnp.min(jnp.where(masked == m2, lane, E), axis=1, keepdims=True)
        denom = m1 + m2 + 1e-20
        w1 = m1 / denom
        w2 = m2 / denom
        lane16 = jax.lax.broadcasted_iota(jnp.int32, (S, 16), 1)
        w9_scr[...] = (jnp.where(lane16 == i1, w1, 0.0)
                       + jnp.where(lane16 == i2, w2, 0.0))

        hsh = _dmlp(x, Wsg_ref[...].astype(jnp.bfloat16),
                    Wsu_ref[...].astype(jnp.bfloat16))
        out_ref[...] = jax.lax.dot_general(
            hsh.astype(jnp.bfloat16), Wsd_ref[...].astype(jnp.bfloat16),
            _DN, preferred_element_type=jnp.float32)

    lane16 = jax.lax.broadcasted_iota(jnp.int32, (S, 16), 1)
    wcol = jnp.sum(jnp.where(lane16 == e, w9_scr[...], 0.0),
                   axis=1, keepdims=True)
    h = _dmlp(xb_scr[...], Wg_ref[0].astype(jnp.bfloat16),
              Wu_ref[0].astype(jnp.bfloat16)) * wcol
    out_ref[...] += jax.lax.dot_general(
        h.astype(jnp.bfloat16), Wd_ref[0].astype(jnp.bfloat16),
        _DN, preferred_element_type=jnp.float32)


@jax.jit
def kernel(hidden_states, g, weight_token, weight_gene,
           Wg, Wu, Wd, Wsg, Wsu, Wsd):
    y = pl.pallas_call(
        _moe_body,
        grid=(E,),
        in_specs=[
            pl.BlockSpec((S, H), lambda e: (0, 0)),
            pl.BlockSpec((1, G), lambda e: (0, 0)),
            pl.BlockSpec((E, H), lambda e: (0, 0)),
            pl.BlockSpec((E, G), lambda e: (0, 0)),
            pl.BlockSpec((I, H), lambda e: (0, 0)),
            pl.BlockSpec((I, H), lambda e: (0, 0)),
            pl.BlockSpec((H, I), lambda e: (0, 0)),
            pl.BlockSpec((1, I, H), lambda e: (e, 0, 0)),
            pl.BlockSpec((1, I, H), lambda e: (e, 0, 0)),
            pl.BlockSpec((1, H, I), lambda e: (e, 0, 0)),
        ],
        out_specs=pl.BlockSpec((S, H), lambda e: (0, 0)),
        out_shape=jax.ShapeDtypeStruct((S, H), jnp.float32),
        scratch_shapes=[pltpu.VMEM((S, 16), jnp.float32),
                        pltpu.VMEM((S, H), jnp.bfloat16)],
        compiler_params=pltpu.CompilerParams(
            dimension_semantics=("arbitrary",)),
    )(hidden_states.reshape(S, H), g, weight_token, weight_gene,
      Wsg, Wsu, Wsd, Wg, Wu, Wd)
    return y.reshape(B, S, H)


# speedup vs baseline: 7.3687x; 1.0165x over previous
import jax
import jax.numpy as jnp
from jax.experimental import pallas as pl
from jax.experimental.pallas import tpu as pltpu

B, S, H = 1, 2048, 1024
E, K = 8, 2
I = 256
G = 128
TEMP = 1.0
PAIR = 2
NP = E // PAIR

_DN = (((1,), (1,)), ((), ()))


def _dmlp(x, wg, wu):
    xg = jax.lax.dot_general(x, wg, _DN, preferred_element_type=jnp.float32)
    xu = jax.lax.dot_general(x, wu, _DN, preferred_element_type=jnp.float32)
    return jnp.where(xg >= 0, xg, 0.01 * xg) * xu


def _moe_body(x_ref, g_ref, wt_ref, wgene_ref, Wsg_ref, Wsu_ref, Wsd_ref,
              Wg_ref, Wu_ref, Wd_ref, out_ref, w9_scr, xb_scr):
    p = pl.program_id(0)

    @pl.when(p == 0)
    def _router_and_shared():
        xb_scr[...] = x_ref[...].astype(jnp.bfloat16)
        x = xb_scr[...]
        logits_h = jax.lax.dot_general(
            x, wt_ref[...].astype(jnp.bfloat16), _DN,
            preferred_element_type=jnp.float32)
        logits_g = jax.lax.dot_general(
            g_ref[...].astype(jnp.bfloat16),
            wgene_ref[...].astype(jnp.bfloat16), _DN,
            preferred_element_type=jnp.float32)
        logits = (logits_h + logits_g / TEMP) / (1.0 + 1.0 / TEMP)
        m = jnp.max(logits, axis=1, keepdims=True)
        ex = jnp.exp(logits - m)
        scores = ex / jnp.sum(ex, axis=1, keepdims=True)
        lane = jax.lax.broadcasted_iota(jnp.int32, (S, E), 1)
        m1 = jnp.max(scores, axis=1, keepdims=True)
        i1 = jnp.min(jnp.where(scores == m1, lane, E), axis=1, keepdims=True)
        masked = jnp.where(lane == i1, -jnp.inf, scores)
        m2 = jnp.max(masked, axis=1, keepdims=True)
        i2 = jnp.min(jnp.where(masked == m2, lane, E), axis=1, keepdims=True)
        denom = m1 + m2 + 1e-20
        w1 = m1 / denom
        w2 = m2 / denom
        lane16 = jax.lax.broadcasted_iota(jnp.int32, (S, 16), 1)
        w9_scr[...] = (jnp.where(lane16 == i1, w1, 0.0)
                       + jnp.where(lane16 == i2, w2, 0.0))

        hsh = _dmlp(x, Wsg_ref[...].astype(jnp.bfloat16),
                    Wsu_ref[...].astype(jnp.bfloat16))
        out_ref[...] = jax.lax.dot_general(
            hsh.astype(jnp.bfloat16), Wsd_ref[...].astype(jnp.bfloat16),
            _DN, preferred_element_type=jnp.float32)

    lane16 = jax.lax.broadcasted_iota(jnp.int32, (S, 16), 1)
    xb = xb_scr[...]
    w9 = w9_scr[...]
    yes = []
    for j in range(PAIR):
        e = p * PAIR + j
        wcol = jnp.sum(jnp.where(lane16 == e, w9, 0.0),
                       axis=1, keepdims=True)
        h = _dmlp(xb, Wg_ref[j].astype(jnp.bfloat16),
                  Wu_ref[j].astype(jnp.bfloat16)) * wcol
        yes.append(jax.lax.dot_general(
            h.astype(jnp.bfloat16), Wd_ref[j].astype(jnp.bfloat16),
            _DN, preferred_element_type=jnp.float32))
    out_ref[...] += yes[0] + yes[1]


@jax.jit
def kernel(hidden_states, g, weight_token, weight_gene,
           Wg, Wu, Wd, Wsg, Wsu, Wsd):
    y = pl.pallas_call(
        _moe_body,
        grid=(NP,),
        in_specs=[
            pl.BlockSpec((S, H), lambda p: (0, 0)),
            pl.BlockSpec((1, G), lambda p: (0, 0)),
            pl.BlockSpec((E, H), lambda p: (0, 0)),
            pl.BlockSpec((E, G), lambda p: (0, 0)),
            pl.BlockSpec((I, H), lambda p: (0, 0)),
            pl.BlockSpec((I, H), lambda p: (0, 0)),
            pl.BlockSpec((H, I), lambda p: (0, 0)),
            pl.BlockSpec((PAIR, I, H), lambda p: (p, 0, 0)),
            pl.BlockSpec((PAIR, I, H), lambda p: (p, 0, 0)),
            pl.BlockSpec((PAIR, H, I), lambda p: (p, 0, 0)),
        ],
        out_specs=pl.BlockSpec((S, H), lambda p: (0, 0)),
        out_shape=jax.ShapeDtypeStruct((S, H), jnp.float32),
        scratch_shapes=[pltpu.VMEM((S, 16), jnp.float32),
                        pltpu.VMEM((S, H), jnp.bfloat16)],
        compiler_params=pltpu.CompilerParams(
            dimension_semantics=("arbitrary",)),
    )(hidden_states.reshape(S, H), g, weight_token, weight_gene,
      Wsg, Wsu, Wsd, Wg, Wu, Wd)
    return y.reshape(B, S, H)
